# baseline (speedup 1.0000x reference)
import functools

import jax
import jax.numpy as jnp
from jax import lax
from jax.experimental import pallas as pl
from jax.experimental.pallas import tpu as pltpu
from jax.experimental.pallas import tpu_sc as plsc

N = 10000
E = 320000
NC, NS = 2, 16
NW = NC * NS
CH = 128
CPT = 80
NBUF = 8
EP = NW * CPT * CH
NP = 10112
RPT = NP // NS
ROW_BLK = 632

_mesh = plsc.VectorSubcoreMesh(
    core_axis_name="c", subcore_axis_name="s", num_cores=NC, num_subcores=NS
)
_sc_params = pltpu.CompilerParams(use_tc_tiling_on_sc=False)


def _deg_body(dst_hbm, ones_hbm, z16_hbm, out_hbm, dst_v, ones_v, acc):
    cid = lax.axis_index("c")
    sid = lax.axis_index("s")
    wid = cid * NS + sid
    pltpu.sync_copy(dst_hbm.at[wid], dst_v)
    pltpu.sync_copy(ones_hbm, ones_v)
    pltpu.sync_copy(z16_hbm, acc.at[pl.ds(sid * RPT, RPT)])
    plsc.subcore_barrier()

    @pl.loop(0, CPT)
    def _(j):
        pltpu.sync_copy(ones_v, acc.at[dst_v.at[j]], add=True)

    plsc.subcore_barrier()
    pltpu.sync_copy(
        acc.at[pl.ds(sid * RPT, RPT)],
        out_hbm.at[cid, pl.ds(sid * RPT, RPT)],
    )


_deg_kernel = functools.partial(
    pl.kernel,
    out_type=jax.ShapeDtypeStruct((NC, NP, 16), jnp.float32),
    mesh=_mesh,
    scratch_types=[
        pltpu.VMEM((CPT, CH), jnp.int32),
        pltpu.VMEM((CH, 16), jnp.float32),
        pltpu.MemorySpace.VMEM_SHARED((NP, 16), jnp.float32),
    ],
    compiler_params=_sc_params,
)(_deg_body)


def _prop_body(D, xs_hbm, src_hbm, dst_hbm, zrows_hbm, out_hbm,
               src_v, dst_v, buf, acc, *sems):
    semg = sems[:NBUF]
    sems_ = sems[NBUF:]
    cid = lax.axis_index("c")
    sid = lax.axis_index("s")
    wid = cid * NS + sid
    pltpu.sync_copy(src_hbm.at[wid], src_v)
    pltpu.sync_copy(dst_hbm.at[wid], dst_v)
    pltpu.sync_copy(zrows_hbm, acc.at[pl.ds(sid * RPT, RPT)])
    plsc.subcore_barrier()

    def gather(j, b):
        return pltpu.make_async_copy(xs_hbm.at[src_v.at[j]], buf.at[b], semg[b])

    def scatter(j, b):
        return pltpu.make_async_copy(buf.at[b], acc.at[dst_v.at[j]], sems_[b])

    for b in range(NBUF):
        gather(b, b).start()

    G = CPT // NBUF

    @pl.loop(0, G - 1)
    def _(k):
        j0 = k * NBUF
        for b in range(NBUF):
            gather(j0 + b, b).wait()
            scatter(j0 + b, b).start(add=True)
        for b in range(NBUF):
            scatter(j0 + b, b).wait()
            gather(j0 + NBUF + b, b).start()

    j0 = (G - 1) * NBUF
    for b in range(NBUF):
        gather(j0 + b, b).wait()
        scatter(j0 + b, b).start(add=True)
    for b in range(NBUF):
        scatter(j0 + b, b).wait()

    plsc.subcore_barrier()
    pltpu.sync_copy(
        acc.at[pl.ds(sid * RPT, RPT)],
        out_hbm.at[cid, pl.ds(sid * RPT, RPT)],
    )


def _make_prop(D):
    return functools.partial(
        pl.kernel,
        out_type=jax.ShapeDtypeStruct((NC, NP, D), jnp.float32),
        mesh=_mesh,
        scratch_types=[
            pltpu.VMEM((CPT, CH), jnp.int32),
            pltpu.VMEM((CPT, CH), jnp.int32),
            pltpu.VMEM((NBUF, CH, D), jnp.float32),
            pltpu.MemorySpace.VMEM_SHARED((NP, D), jnp.float32),
        ]
        + [pltpu.SemaphoreType.DMA] * (2 * NBUF),
        compiler_params=_sc_params,
    )(functools.partial(_prop_body, D))


_prop64 = _make_prop(64)
_prop16 = _make_prop(16)


def _tc_first_body(x_ref, w_ref, dega_ref, degb_ref, xs_ref, dis_ref):
    deg = dega_ref[...] + degb_ref[...] + 1.0
    dis = lax.rsqrt(deg)
    xw = jnp.dot(x_ref[...], w_ref[...], preferred_element_type=jnp.float32)
    xs_ref[...] = xw * dis
    dis_ref[...] = dis


def _tc_first(x, w1, dega, degb):
    return pl.pallas_call(
        _tc_first_body,
        grid=(NP // ROW_BLK,),
        in_specs=[
            pl.BlockSpec((ROW_BLK, 128), lambda i: (i, 0)),
            pl.BlockSpec((128, 64), lambda i: (0, 0)),
            pl.BlockSpec((ROW_BLK, 1), lambda i: (i, 0)),
            pl.BlockSpec((ROW_BLK, 1), lambda i: (i, 0)),
        ],
        out_specs=[
            pl.BlockSpec((ROW_BLK, 64), lambda i: (i, 0)),
            pl.BlockSpec((ROW_BLK, 1), lambda i: (i, 0)),
        ],
        out_shape=[
            jax.ShapeDtypeStruct((NP, 64), jnp.float32),
            jax.ShapeDtypeStruct((NP, 1), jnp.float32),
        ],
    )(x, w1, dega, degb)


def _tc_mid_body(sa_ref, sb_ref, xs_ref, dis_ref, b_ref, w_ref, out_ref):
    dis = dis_ref[...]
    h = dis * (sa_ref[...] + sb_ref[...] + xs_ref[...]) + b_ref[...]
    h = jnp.maximum(h, 0.0)
    xw = jnp.dot(h, w_ref[...], preferred_element_type=jnp.float32)
    out_ref[...] = xw * dis


def _tc_mid(sa, sb, xs, dis, b, w):
    dout = w.shape[1]
    return pl.pallas_call(
        _tc_mid_body,
        grid=(NP // ROW_BLK,),
        in_specs=[
            pl.BlockSpec((ROW_BLK, 64), lambda i: (i, 0)),
            pl.BlockSpec((ROW_BLK, 64), lambda i: (i, 0)),
            pl.BlockSpec((ROW_BLK, 64), lambda i: (i, 0)),
            pl.BlockSpec((ROW_BLK, 1), lambda i: (i, 0)),
            pl.BlockSpec((1, 64), lambda i: (0, 0)),
            pl.BlockSpec((64, dout), lambda i: (0, 0)),
        ],
        out_specs=pl.BlockSpec((ROW_BLK, dout), lambda i: (i, 0)),
        out_shape=jax.ShapeDtypeStruct((NP, dout), jnp.float32),
    )(sa, sb, xs, dis, b, w)


def _tc_last_body(sa_ref, sb_ref, xs_ref, dis_ref, b_ref, out_ref):
    s = dis_ref[...] * (sa_ref[...] + sb_ref[...] + xs_ref[...])
    out_ref[...] = s[:, :4] + b_ref[...]


def _tc_last(sa, sb, xs, dis, b3):
    return pl.pallas_call(
        _tc_last_body,
        grid=(NP // ROW_BLK,),
        in_specs=[
            pl.BlockSpec((ROW_BLK, 16), lambda i: (i, 0)),
            pl.BlockSpec((ROW_BLK, 16), lambda i: (i, 0)),
            pl.BlockSpec((ROW_BLK, 16), lambda i: (i, 0)),
            pl.BlockSpec((ROW_BLK, 1), lambda i: (i, 0)),
            pl.BlockSpec((1, 4), lambda i: (0, 0)),
        ],
        out_specs=pl.BlockSpec((ROW_BLK, 4), lambda i: (i, 0)),
        out_shape=jax.ShapeDtypeStruct((NP, 4), jnp.float32),
    )(sa, sb, xs, dis, b3)


def kernel(x, edge_index, W1, b1, W2, b2, W3, b3):
    ei = edge_index.astype(jnp.int32)
    pad = EP - E
    padv = N + jnp.arange(pad, dtype=jnp.int32) % (NP - N)
    src = jnp.concatenate([ei[0], padv])
    dst = jnp.concatenate([ei[1], padv])
    src_r = src.reshape(NW, CPT, CH)
    dst_r = dst.reshape(NW, CPT, CH)

    ones128 = jnp.ones((CH, 16), jnp.float32)
    z16 = jnp.zeros((RPT, 16), jnp.float32)
    z64 = jnp.zeros((RPT, 64), jnp.float32)

    xp = jnp.pad(x, ((0, NP - N), (0, 0)))
    degp = _deg_kernel(dst_r, ones128, z16)

    xs1, dis = _tc_first(xp, W1, degp[0, :, :1], degp[1, :, :1])
    s1 = _prop64(xs1, src_r, dst_r, z64)

    xs2 = _tc_mid(s1[0], s1[1], xs1, dis, b1.reshape(1, 64), W2)
    s2 = _prop64(xs2, src_r, dst_r, z64)

    w3p = jnp.pad(W3, ((0, 0), (0, 16 - W3.shape[1])))
    xs3 = _tc_mid(s2[0], s2[1], xs2, dis, b2.reshape(1, 64), w3p)
    s3 = _prop16(xs3, src_r, dst_r, z16)

    return _tc_last(s3[0], s3[1], xs3, dis, b3.reshape(1, 4))[:N]


# speedup vs baseline: 1.0757x; 1.0757x over previous
import functools

import jax
import jax.numpy as jnp
from jax import lax
from jax.experimental import pallas as pl
from jax.experimental.pallas import tpu as pltpu
from jax.experimental.pallas import tpu_sc as plsc

N = 10000
E = 320000
NC, NS = 2, 16
NW = NC * NS
CH = 128
CPT = 80
NBUF = 8
EP = NW * CPT * CH
NP = 10112
RPT = NP // NS
ROW_BLK = 2528

_mesh = plsc.VectorSubcoreMesh(
    core_axis_name="c", subcore_axis_name="s", num_cores=NC, num_subcores=NS
)
_sc_params = pltpu.CompilerParams(use_tc_tiling_on_sc=False)


def _deg_body(dst_hbm, ones_hbm, z16_hbm, out_hbm, dst_v, ones_v, acc):
    cid = lax.axis_index("c")
    sid = lax.axis_index("s")
    wid = cid * NS + sid
    pltpu.sync_copy(dst_hbm.at[wid], dst_v)
    pltpu.sync_copy(ones_hbm, ones_v)
    pltpu.sync_copy(z16_hbm, acc.at[pl.ds(sid * RPT, RPT)])
    plsc.subcore_barrier()

    @pl.loop(0, CPT)
    def _(j):
        pltpu.sync_copy(ones_v, acc.at[dst_v.at[j]], add=True)

    plsc.subcore_barrier()
    pltpu.sync_copy(
        acc.at[pl.ds(sid * RPT, RPT)],
        out_hbm.at[cid, pl.ds(sid * RPT, RPT)],
    )


_deg_kernel = functools.partial(
    pl.kernel,
    out_type=jax.ShapeDtypeStruct((NC, NP, 16), jnp.float32),
    mesh=_mesh,
    scratch_types=[
        pltpu.VMEM((CPT, CH), jnp.int32),
        pltpu.VMEM((CH, 16), jnp.float32),
        pltpu.MemorySpace.VMEM_SHARED((NP, 16), jnp.float32),
    ],
    compiler_params=_sc_params,
)(_deg_body)


def _prop_body(D, xs_hbm, src_hbm, dst_hbm, zrows_hbm, out_hbm,
               src_v, dst_v, buf, acc, *sems):
    semg = sems[:NBUF]
    sems_ = sems[NBUF:]
    cid = lax.axis_index("c")
    sid = lax.axis_index("s")
    wid = cid * NS + sid
    pltpu.sync_copy(src_hbm.at[wid], src_v)
    pltpu.sync_copy(dst_hbm.at[wid], dst_v)
    pltpu.sync_copy(zrows_hbm, acc.at[pl.ds(sid * RPT, RPT)])
    plsc.subcore_barrier()

    def gather(j, b):
        return pltpu.make_async_copy(xs_hbm.at[src_v.at[j]], buf.at[b], semg[b])

    def scatter(j, b):
        return pltpu.make_async_copy(buf.at[b], acc.at[dst_v.at[j]], sems_[b])

    for b in range(NBUF):
        gather(b, b).start()

    G = CPT // NBUF

    @pl.loop(0, G - 1)
    def _(k):
        j0 = k * NBUF
        for b in range(NBUF):
            gather(j0 + b, b).wait()
            scatter(j0 + b, b).start(add=True)
        for b in range(NBUF):
            scatter(j0 + b, b).wait()
            gather(j0 + NBUF + b, b).start()

    j0 = (G - 1) * NBUF
    for b in range(NBUF):
        gather(j0 + b, b).wait()
        scatter(j0 + b, b).start(add=True)
    for b in range(NBUF):
        scatter(j0 + b, b).wait()

    plsc.subcore_barrier()
    pltpu.sync_copy(
        acc.at[pl.ds(sid * RPT, RPT)],
        out_hbm.at[cid, pl.ds(sid * RPT, RPT)],
    )


def _make_prop(D):
    return functools.partial(
        pl.kernel,
        out_type=jax.ShapeDtypeStruct((NC, NP, D), jnp.float32),
        mesh=_mesh,
        scratch_types=[
            pltpu.VMEM((CPT, CH), jnp.int32),
            pltpu.VMEM((CPT, CH), jnp.int32),
            pltpu.VMEM((NBUF, CH, D), jnp.float32),
            pltpu.MemorySpace.VMEM_SHARED((NP, D), jnp.float32),
        ]
        + [pltpu.SemaphoreType.DMA] * (2 * NBUF),
        compiler_params=_sc_params,
    )(functools.partial(_prop_body, D))


_prop64 = _make_prop(64)
_prop16 = _make_prop(16)


def _tc_first_body(x_ref, w_ref, dega_ref, degb_ref, xs_ref, dis_ref):
    deg = dega_ref[...] + degb_ref[...] + 1.0
    dis = lax.rsqrt(deg)
    xw = jnp.dot(x_ref[...], w_ref[...], preferred_element_type=jnp.float32)
    xs_ref[...] = xw * dis
    dis_ref[...] = dis


def _tc_first(x, w1, dega, degb):
    return pl.pallas_call(
        _tc_first_body,
        grid=(NP // ROW_BLK,),
        in_specs=[
            pl.BlockSpec((ROW_BLK, 128), lambda i: (i, 0)),
            pl.BlockSpec((128, 64), lambda i: (0, 0)),
            pl.BlockSpec((ROW_BLK, 1), lambda i: (i, 0)),
            pl.BlockSpec((ROW_BLK, 1), lambda i: (i, 0)),
        ],
        out_specs=[
            pl.BlockSpec((ROW_BLK, 64), lambda i: (i, 0)),
            pl.BlockSpec((ROW_BLK, 1), lambda i: (i, 0)),
        ],
        out_shape=[
            jax.ShapeDtypeStruct((NP, 64), jnp.float32),
            jax.ShapeDtypeStruct((NP, 1), jnp.float32),
        ],
    )(x, w1, dega, degb)


def _tc_mid_body(sa_ref, sb_ref, xs_ref, dis_ref, b_ref, w_ref, out_ref):
    dis = dis_ref[...]
    h = dis * (sa_ref[...] + sb_ref[...] + xs_ref[...]) + b_ref[...]
    h = jnp.maximum(h, 0.0)
    xw = jnp.dot(h, w_ref[...], preferred_element_type=jnp.float32)
    out_ref[...] = xw * dis


def _tc_mid(sa, sb, xs, dis, b, w):
    dout = w.shape[1]
    return pl.pallas_call(
        _tc_mid_body,
        grid=(NP // ROW_BLK,),
        in_specs=[
            pl.BlockSpec((ROW_BLK, 64), lambda i: (i, 0)),
            pl.BlockSpec((ROW_BLK, 64), lambda i: (i, 0)),
            pl.BlockSpec((ROW_BLK, 64), lambda i: (i, 0)),
            pl.BlockSpec((ROW_BLK, 1), lambda i: (i, 0)),
            pl.BlockSpec((1, 64), lambda i: (0, 0)),
            pl.BlockSpec((64, dout), lambda i: (0, 0)),
        ],
        out_specs=pl.BlockSpec((ROW_BLK, dout), lambda i: (i, 0)),
        out_shape=jax.ShapeDtypeStruct((NP, dout), jnp.float32),
    )(sa, sb, xs, dis, b, w)


def _tc_last_body(sa_ref, sb_ref, xs_ref, dis_ref, b_ref, out_ref):
    s = dis_ref[...] * (sa_ref[...] + sb_ref[...] + xs_ref[...])
    out_ref[...] = s[:, :4] + b_ref[...]


def _tc_last(sa, sb, xs, dis, b3):
    return pl.pallas_call(
        _tc_last_body,
        grid=(NP // ROW_BLK,),
        in_specs=[
            pl.BlockSpec((ROW_BLK, 16), lambda i: (i, 0)),
            pl.BlockSpec((ROW_BLK, 16), lambda i: (i, 0)),
            pl.BlockSpec((ROW_BLK, 16), lambda i: (i, 0)),
            pl.BlockSpec((ROW_BLK, 1), lambda i: (i, 0)),
            pl.BlockSpec((1, 4), lambda i: (0, 0)),
        ],
        out_specs=pl.BlockSpec((ROW_BLK, 4), lambda i: (i, 0)),
        out_shape=jax.ShapeDtypeStruct((NP, 4), jnp.float32),
    )(sa, sb, xs, dis, b3)


def kernel(x, edge_index, W1, b1, W2, b2, W3, b3):
    ei = edge_index.astype(jnp.int32)
    pad = EP - E
    padv = N + jnp.arange(pad, dtype=jnp.int32) % (NP - N)
    src = jnp.concatenate([ei[0], padv])
    dst = jnp.concatenate([ei[1], padv])
    src_r = src.reshape(NW, CPT, CH)
    dst_r = dst.reshape(NW, CPT, CH)

    ones128 = jnp.ones((CH, 16), jnp.float32)
    z16 = jnp.zeros((RPT, 16), jnp.float32)
    z64 = jnp.zeros((RPT, 64), jnp.float32)

    xp = jnp.pad(x, ((0, NP - N), (0, 0)))
    degp = _deg_kernel(dst_r, ones128, z16)

    xs1, dis = _tc_first(xp, W1, degp[0, :, :1], degp[1, :, :1])
    s1 = _prop64(xs1, src_r, dst_r, z64)

    xs2 = _tc_mid(s1[0], s1[1], xs1, dis, b1.reshape(1, 64), W2)
    s2 = _prop64(xs2, src_r, dst_r, z64)

    w3p = jnp.pad(W3, ((0, 0), (0, 16 - W3.shape[1])))
    xs3 = _tc_mid(s2[0], s2[1], xs2, dis, b2.reshape(1, 64), w3p)
    s3 = _prop16(xs3, src_r, dst_r, z16)

    return _tc_last(s3[0], s3[1], xs3, dis, b3.reshape(1, 4))[:N]


# speedup vs baseline: 1.1199x; 1.0411x over previous
import functools

import jax
import jax.numpy as jnp
from jax import lax
from jax.experimental import pallas as pl
from jax.experimental.pallas import tpu as pltpu
from jax.experimental.pallas import tpu_sc as plsc

N = 10000
E = 320000
NC, NS = 2, 16
NW = NC * NS
CH = 125
CPT = 80
NBUF = 8
NCHUNK = NW * CPT
NP = 10112
RPT = NP // NS
ROW_BLK = 2528

_mesh = plsc.VectorSubcoreMesh(
    core_axis_name="c", subcore_axis_name="s", num_cores=NC, num_subcores=NS
)
_sc_params = pltpu.CompilerParams(use_tc_tiling_on_sc=False)


def _deg_body(er_hbm, ones_hbm, z16_hbm, out_hbm, dst_v, ones_v, acc, sem):
    cid = lax.axis_index("c")
    sid = lax.axis_index("s")
    wid = cid * NS + sid
    pltpu.sync_copy(er_hbm.at[1, pl.ds(wid * CPT, CPT)], dst_v)
    pltpu.sync_copy(ones_hbm, ones_v)
    pltpu.sync_copy(z16_hbm, acc.at[pl.ds(sid * RPT, RPT)])
    plsc.subcore_barrier()

    def scat(j):
        return pltpu.make_async_copy(ones_v, acc.at[dst_v.at[j]], sem)

    @pl.loop(0, CPT)
    def _(j):
        scat(j).start(add=True)

    @pl.loop(0, CPT)
    def _(j):
        scat(j).wait()

    plsc.subcore_barrier()
    pltpu.sync_copy(
        acc.at[pl.ds(sid * RPT, RPT)],
        out_hbm.at[cid, pl.ds(sid * RPT, RPT)],
    )


_deg_kernel = functools.partial(
    pl.kernel,
    out_type=jax.ShapeDtypeStruct((NC, NP, 16), jnp.float32),
    mesh=_mesh,
    scratch_types=[
        pltpu.VMEM((CPT, CH), jnp.int32),
        pltpu.VMEM((CH, 16), jnp.float32),
        pltpu.MemorySpace.VMEM_SHARED((NP, 16), jnp.float32),
        pltpu.SemaphoreType.DMA,
    ],
    compiler_params=_sc_params,
)(_deg_body)


def _prop_body(D, xs_hbm, er_hbm, zrows_hbm, out_hbm,
               src_v, dst_v, buf, acc, *sems):
    semg = sems[:NBUF]
    sems_ = sems[NBUF:]
    cid = lax.axis_index("c")
    sid = lax.axis_index("s")
    wid = cid * NS + sid
    pltpu.sync_copy(er_hbm.at[0, pl.ds(wid * CPT, CPT)], src_v)
    pltpu.sync_copy(er_hbm.at[1, pl.ds(wid * CPT, CPT)], dst_v)
    pltpu.sync_copy(zrows_hbm, acc.at[pl.ds(sid * RPT, RPT)])
    plsc.subcore_barrier()

    def gather(j, b):
        return pltpu.make_async_copy(xs_hbm.at[src_v.at[j]], buf.at[b], semg[b])

    def scatter(j, b):
        return pltpu.make_async_copy(buf.at[b], acc.at[dst_v.at[j]], sems_[b])

    for b in range(NBUF):
        gather(b, b).start()

    G = CPT // NBUF

    @pl.loop(0, G - 1)
    def _(k):
        j0 = k * NBUF
        for b in range(NBUF):
            gather(j0 + b, b).wait()
            scatter(j0 + b, b).start(add=True)
        for b in range(NBUF):
            scatter(j0 + b, b).wait()
            gather(j0 + NBUF + b, b).start()

    j0 = (G - 1) * NBUF
    for b in range(NBUF):
        gather(j0 + b, b).wait()
        scatter(j0 + b, b).start(add=True)
    for b in range(NBUF):
        scatter(j0 + b, b).wait()

    plsc.subcore_barrier()
    pltpu.sync_copy(
        acc.at[pl.ds(sid * RPT, RPT)],
        out_hbm.at[cid, pl.ds(sid * RPT, RPT)],
    )


def _make_prop(D):
    return functools.partial(
        pl.kernel,
        out_type=jax.ShapeDtypeStruct((NC, NP, D), jnp.float32),
        mesh=_mesh,
        scratch_types=[
            pltpu.VMEM((CPT, CH), jnp.int32),
            pltpu.VMEM((CPT, CH), jnp.int32),
            pltpu.VMEM((NBUF, CH, D), jnp.float32),
            pltpu.MemorySpace.VMEM_SHARED((NP, D), jnp.float32),
        ]
        + [pltpu.SemaphoreType.DMA] * (2 * NBUF),
        compiler_params=_sc_params,
    )(functools.partial(_prop_body, D))


_prop64 = _make_prop(64)
_prop16 = _make_prop(16)


def _tc_first_body(x_ref, w_ref, dega_ref, degb_ref, xs_ref, dis_ref):
    deg = dega_ref[...] + degb_ref[...] + 1.0
    dis = lax.rsqrt(deg)
    xw = jnp.dot(x_ref[...], w_ref[...], preferred_element_type=jnp.float32)
    xs_ref[...] = xw * dis
    dis_ref[...] = dis


def _tc_first(x, w1, dega, degb):
    return pl.pallas_call(
        _tc_first_body,
        grid=(NP // ROW_BLK,),
        in_specs=[
            pl.BlockSpec((ROW_BLK, 128), lambda i: (i, 0)),
            pl.BlockSpec((128, 64), lambda i: (0, 0)),
            pl.BlockSpec((ROW_BLK, 1), lambda i: (i, 0)),
            pl.BlockSpec((ROW_BLK, 1), lambda i: (i, 0)),
        ],
        out_specs=[
            pl.BlockSpec((ROW_BLK, 64), lambda i: (i, 0)),
            pl.BlockSpec((ROW_BLK, 1), lambda i: (i, 0)),
        ],
        out_shape=[
            jax.ShapeDtypeStruct((NP, 64), jnp.float32),
            jax.ShapeDtypeStruct((NP, 1), jnp.float32),
        ],
    )(x, w1, dega, degb)


def _tc_mid_body(sa_ref, sb_ref, xs_ref, dis_ref, b_ref, w_ref, out_ref):
    dis = dis_ref[...]
    h = dis * (sa_ref[...] + sb_ref[...] + xs_ref[...]) + b_ref[...]
    h = jnp.maximum(h, 0.0)
    xw = jnp.dot(h, w_ref[...], preferred_element_type=jnp.float32)
    out_ref[...] = xw * dis


def _tc_mid(sa, sb, xs, dis, b, w):
    dout = w.shape[1]
    return pl.pallas_call(
        _tc_mid_body,
        grid=(NP // ROW_BLK,),
        in_specs=[
            pl.BlockSpec((ROW_BLK, 64), lambda i: (i, 0)),
            pl.BlockSpec((ROW_BLK, 64), lambda i: (i, 0)),
            pl.BlockSpec((ROW_BLK, 64), lambda i: (i, 0)),
            pl.BlockSpec((ROW_BLK, 1), lambda i: (i, 0)),
            pl.BlockSpec((1, 64), lambda i: (0, 0)),
            pl.BlockSpec((64, dout), lambda i: (0, 0)),
        ],
        out_specs=pl.BlockSpec((ROW_BLK, dout), lambda i: (i, 0)),
        out_shape=jax.ShapeDtypeStruct((NP, dout), jnp.float32),
    )(sa, sb, xs, dis, b, w)


def _tc_last_body(sa_ref, sb_ref, xs_ref, dis_ref, b_ref, out_ref):
    s = dis_ref[...] * (sa_ref[...] + sb_ref[...] + xs_ref[...])
    out_ref[...] = s[:, :4] + b_ref[...]


def _tc_last(sa, sb, xs, dis, b3):
    return pl.pallas_call(
        _tc_last_body,
        grid=(NP // ROW_BLK,),
        in_specs=[
            pl.BlockSpec((ROW_BLK, 16), lambda i: (i, 0)),
            pl.BlockSpec((ROW_BLK, 16), lambda i: (i, 0)),
            pl.BlockSpec((ROW_BLK, 16), lambda i: (i, 0)),
            pl.BlockSpec((ROW_BLK, 1), lambda i: (i, 0)),
            pl.BlockSpec((1, 4), lambda i: (0, 0)),
        ],
        out_specs=pl.BlockSpec((ROW_BLK, 4), lambda i: (i, 0)),
        out_shape=jax.ShapeDtypeStruct((NP, 4), jnp.float32),
    )(sa, sb, xs, dis, b3)


def kernel(x, edge_index, W1, b1, W2, b2, W3, b3):
    er = edge_index.astype(jnp.int32).reshape(2, NCHUNK, CH)

    ones125 = jnp.ones((CH, 16), jnp.float32)
    z16 = jnp.zeros((RPT, 16), jnp.float32)
    z64 = jnp.zeros((RPT, 64), jnp.float32)

    xp = jnp.pad(x, ((0, NP - N), (0, 0)))
    degp = _deg_kernel(er, ones125, z16)

    xs1, dis = _tc_first(xp, W1, degp[0, :, :1], degp[1, :, :1])
    s1 = _prop64(xs1, er, z64)

    xs2 = _tc_mid(s1[0], s1[1], xs1, dis, b1.reshape(1, 64), W2)
    s2 = _prop64(xs2, er, z64)

    w3p = jnp.pad(W3, ((0, 0), (0, 16 - W3.shape[1])))
    xs3 = _tc_mid(s2[0], s2[1], xs2, dis, b2.reshape(1, 64), w3p)
    s3 = _prop16(xs3, er, z16)

    return _tc_last(s3[0], s3[1], xs3, dis, b3.reshape(1, 4))[:N]


# speedup vs baseline: 1.2985x; 1.1594x over previous
import functools

import jax
import jax.numpy as jnp
from jax import lax
from jax.experimental import pallas as pl
from jax.experimental.pallas import tpu as pltpu
from jax.experimental.pallas import tpu_sc as plsc

N = 10000
E = 320000
NC, NS = 2, 16
NW = NC * NS
CH = 125
CPT = 80
NBUF = 8
NCHUNK = NW * CPT
NP = 10112
RPT = NP // NS
ROW_BLK = 2528

_mesh = plsc.VectorSubcoreMesh(
    core_axis_name="c", subcore_axis_name="s", num_cores=NC, num_subcores=NS
)
_sc_params = pltpu.CompilerParams(use_tc_tiling_on_sc=False)


def _deg_body(er_hbm, ones_hbm, z16_hbm, out_hbm, dst_v, ones_v, acc, sem):
    cid = lax.axis_index("c")
    sid = lax.axis_index("s")
    wid = cid * NS + sid
    pltpu.sync_copy(er_hbm.at[1, pl.ds(wid * CPT, CPT)], dst_v)
    pltpu.sync_copy(ones_hbm, ones_v)
    pltpu.sync_copy(z16_hbm, acc.at[pl.ds(sid * RPT, RPT)])
    plsc.subcore_barrier()

    def scat(j):
        return pltpu.make_async_copy(ones_v, acc.at[dst_v.at[j]], sem)

    @pl.loop(0, CPT)
    def _(j):
        scat(j).start(add=True)

    @pl.loop(0, CPT)
    def _(j):
        scat(j).wait()

    plsc.subcore_barrier()
    pltpu.sync_copy(
        acc.at[pl.ds(sid * RPT, RPT)],
        out_hbm.at[cid, pl.ds(sid * RPT, RPT)],
    )


_deg_kernel = functools.partial(
    pl.kernel,
    out_type=jax.ShapeDtypeStruct((NC, NP, 16), jnp.float32),
    mesh=_mesh,
    scratch_types=[
        pltpu.VMEM((CPT, CH), jnp.int32),
        pltpu.VMEM((CH, 16), jnp.float32),
        pltpu.MemorySpace.VMEM_SHARED((NP, 16), jnp.float32),
        pltpu.SemaphoreType.DMA,
    ],
    compiler_params=_sc_params,
)(_deg_body)


def _prop_body(D, xs_hbm, er_hbm, zrows_hbm, out_hbm,
               src_v, dst_v, buf, acc, *sems):
    semg = sems[:NBUF]
    sems_ = sems[NBUF:]
    cid = lax.axis_index("c")
    sid = lax.axis_index("s")
    wid = cid * NS + sid
    pltpu.sync_copy(er_hbm.at[0, pl.ds(wid * CPT, CPT)], src_v)
    pltpu.sync_copy(er_hbm.at[1, pl.ds(wid * CPT, CPT)], dst_v)
    pltpu.sync_copy(zrows_hbm, acc.at[pl.ds(sid * RPT, RPT)])
    plsc.subcore_barrier()

    def gather(j, b):
        return pltpu.make_async_copy(xs_hbm.at[src_v.at[j]], buf.at[b], semg[b])

    def scatter(j, b):
        return pltpu.make_async_copy(buf.at[b], acc.at[dst_v.at[j]], sems_[b])

    for b in range(NBUF):
        gather(b, b).start()

    G = CPT // NBUF

    @pl.loop(0, G - 1)
    def _(k):
        j0 = k * NBUF
        for b in range(NBUF):
            gather(j0 + b, b).wait()
            scatter(j0 + b, b).start(add=True)
        for b in range(NBUF):
            scatter(j0 + b, b).wait()
            gather(j0 + NBUF + b, b).start()

    j0 = (G - 1) * NBUF
    for b in range(NBUF):
        gather(j0 + b, b).wait()
        scatter(j0 + b, b).start(add=True)
    for b in range(NBUF):
        scatter(j0 + b, b).wait()

    plsc.subcore_barrier()
    pltpu.sync_copy(
        acc.at[pl.ds(sid * RPT, RPT)],
        out_hbm.at[pl.ds(sid * RPT, RPT), pl.ds(cid * D, D)],
    )


def _make_prop(D):
    return functools.partial(
        pl.kernel,
        out_type=jax.ShapeDtypeStruct((NP, 2 * D), jnp.float32),
        mesh=_mesh,
        scratch_types=[
            pltpu.VMEM((CPT, CH), jnp.int32),
            pltpu.VMEM((CPT, CH), jnp.int32),
            pltpu.VMEM((NBUF, CH, D), jnp.float32),
            pltpu.MemorySpace.VMEM_SHARED((NP, D), jnp.float32),
        ]
        + [pltpu.SemaphoreType.DMA] * (2 * NBUF),
        compiler_params=_sc_params,
    )(functools.partial(_prop_body, D))


_prop64 = _make_prop(64)
_prop16 = _make_prop(16)


def _tc_first_body(x_ref, w_ref, dega_ref, degb_ref, xs_ref, dis_ref):
    deg = dega_ref[...] + degb_ref[...] + 1.0
    dis = lax.rsqrt(deg)
    xw = jnp.dot(x_ref[...], w_ref[...], preferred_element_type=jnp.float32)
    xs_ref[...] = xw * dis
    dis_ref[...] = dis


def _tc_first(x, w1, dega, degb):
    return pl.pallas_call(
        _tc_first_body,
        grid=(NP // ROW_BLK,),
        in_specs=[
            pl.BlockSpec((ROW_BLK, 128), lambda i: (i, 0)),
            pl.BlockSpec((128, 64), lambda i: (0, 0)),
            pl.BlockSpec((ROW_BLK, 1), lambda i: (i, 0)),
            pl.BlockSpec((ROW_BLK, 1), lambda i: (i, 0)),
        ],
        out_specs=[
            pl.BlockSpec((ROW_BLK, 64), lambda i: (i, 0)),
            pl.BlockSpec((ROW_BLK, 1), lambda i: (i, 0)),
        ],
        out_shape=[
            jax.ShapeDtypeStruct((NP, 64), jnp.float32),
            jax.ShapeDtypeStruct((NP, 1), jnp.float32),
        ],
    )(x, w1, dega, degb)


def _tc_mid_body(s_ref, xs_ref, dis_ref, b_ref, w_ref, out_ref):
    dis = dis_ref[...]
    s = s_ref[...]
    h = dis * (s[:, :64] + s[:, 64:] + xs_ref[...]) + b_ref[...]
    h = jnp.maximum(h, 0.0)
    xw = jnp.dot(h, w_ref[...], preferred_element_type=jnp.float32)
    out_ref[...] = xw * dis


def _tc_mid(s, xs, dis, b, w):
    dout = w.shape[1]
    return pl.pallas_call(
        _tc_mid_body,
        grid=(NP // ROW_BLK,),
        in_specs=[
            pl.BlockSpec((ROW_BLK, 128), lambda i: (i, 0)),
            pl.BlockSpec((ROW_BLK, 64), lambda i: (i, 0)),
            pl.BlockSpec((ROW_BLK, 1), lambda i: (i, 0)),
            pl.BlockSpec((1, 64), lambda i: (0, 0)),
            pl.BlockSpec((64, dout), lambda i: (0, 0)),
        ],
        out_specs=pl.BlockSpec((ROW_BLK, dout), lambda i: (i, 0)),
        out_shape=jax.ShapeDtypeStruct((NP, dout), jnp.float32),
    )(s, xs, dis, b, w)


def _tc_last_body(s_ref, xs_ref, dis_ref, b_ref, out_ref):
    s = s_ref[...]
    o = dis_ref[...] * (s[:, :16] + s[:, 16:] + xs_ref[...])
    out_ref[...] = o[:, :4] + b_ref[...]


def _tc_last(s, xs, dis, b3):
    return pl.pallas_call(
        _tc_last_body,
        grid=(NP // ROW_BLK,),
        in_specs=[
            pl.BlockSpec((ROW_BLK, 32), lambda i: (i, 0)),
            pl.BlockSpec((ROW_BLK, 16), lambda i: (i, 0)),
            pl.BlockSpec((ROW_BLK, 1), lambda i: (i, 0)),
            pl.BlockSpec((1, 4), lambda i: (0, 0)),
        ],
        out_specs=pl.BlockSpec((ROW_BLK, 4), lambda i: (i, 0)),
        out_shape=jax.ShapeDtypeStruct((NP, 4), jnp.float32),
    )(s, xs, dis, b3)


def kernel(x, edge_index, W1, b1, W2, b2, W3, b3):
    er = edge_index.astype(jnp.int32).reshape(2, NCHUNK, CH)

    ones125 = jnp.ones((CH, 16), jnp.float32)
    z16 = jnp.zeros((RPT, 16), jnp.float32)
    z64 = jnp.zeros((RPT, 64), jnp.float32)

    xp = jnp.pad(x, ((0, NP - N), (0, 0)))
    degp = _deg_kernel(er, ones125, z16)

    xs1, dis = _tc_first(xp, W1, degp[0, :, :1], degp[1, :, :1])
    s1 = _prop64(xs1, er, z64)

    xs2 = _tc_mid(s1, xs1, dis, b1.reshape(1, 64), W2)
    s2 = _prop64(xs2, er, z64)

    w3p = jnp.pad(W3, ((0, 0), (0, 16 - W3.shape[1])))
    xs3 = _tc_mid(s2, xs2, dis, b2.reshape(1, 64), w3p)
    s3 = _prop16(xs3, er, z16)

    return _tc_last(s3, xs3, dis, b3.reshape(1, 4))[:N]


# speedup vs baseline: 1.3481x; 1.0382x over previous
import functools

import jax
import jax.numpy as jnp
from jax import lax
from jax.experimental import pallas as pl
from jax.experimental.pallas import tpu as pltpu
from jax.experimental.pallas import tpu_sc as plsc

N = 10000
E = 320000
NC, NS = 2, 16
NW = NC * NS
CH = 125
CPT = 80
NBUF = 8
NCHUNK = NW * CPT
NP = 10112
RPT = NP // NS
ROW_BLK = 2528

_mesh = plsc.VectorSubcoreMesh(
    core_axis_name="c", subcore_axis_name="s", num_cores=NC, num_subcores=NS
)
_sc_params = pltpu.CompilerParams(use_tc_tiling_on_sc=False)


def _deg_body(er_hbm, ones_hbm, z16_hbm, out_hbm, dst_v, ones_v, acc, sem):
    cid = lax.axis_index("c")
    sid = lax.axis_index("s")
    wid = cid * NS + sid
    pltpu.sync_copy(er_hbm.at[1, pl.ds(wid * CPT, CPT)], dst_v)
    pltpu.sync_copy(ones_hbm, ones_v)
    pltpu.sync_copy(z16_hbm, acc.at[pl.ds(sid * RPT, RPT)])
    plsc.subcore_barrier()

    def scat(j):
        return pltpu.make_async_copy(ones_v, acc.at[dst_v.at[j]], sem)

    @pl.loop(0, CPT)
    def _(j):
        scat(j).start(add=True)

    @pl.loop(0, CPT)
    def _(j):
        scat(j).wait()

    plsc.subcore_barrier()
    pltpu.sync_copy(
        acc.at[pl.ds(sid * RPT, RPT)],
        out_hbm.at[pl.ds(sid * RPT, RPT), pl.ds(cid * 16, 16)],
    )


_deg_kernel = functools.partial(
    pl.kernel,
    out_type=jax.ShapeDtypeStruct((NP, 32), jnp.float32),
    mesh=_mesh,
    scratch_types=[
        pltpu.VMEM((CPT, CH), jnp.int32),
        pltpu.VMEM((CH, 16), jnp.float32),
        pltpu.MemorySpace.VMEM_SHARED((NP, 16), jnp.float32),
        pltpu.SemaphoreType.DMA,
    ],
    compiler_params=_sc_params,
)(_deg_body)


def _prop_body(D, xs_hbm, er_hbm, zrows_hbm, out_hbm,
               src_v, dst_v, buf, acc, *sems):
    semg = sems[:NBUF]
    sems_ = sems[NBUF:]
    cid = lax.axis_index("c")
    sid = lax.axis_index("s")
    wid = cid * NS + sid
    pltpu.sync_copy(er_hbm.at[0, pl.ds(wid * CPT, CPT)], src_v)
    pltpu.sync_copy(er_hbm.at[1, pl.ds(wid * CPT, CPT)], dst_v)
    pltpu.sync_copy(zrows_hbm, acc.at[pl.ds(sid * RPT, RPT)])
    plsc.subcore_barrier()

    def gather(j, b):
        return pltpu.make_async_copy(xs_hbm.at[src_v.at[j]], buf.at[b], semg[b])

    def scatter(j, b):
        return pltpu.make_async_copy(buf.at[b], acc.at[dst_v.at[j]], sems_[b])

    for b in range(NBUF):
        gather(b, b).start()

    G = CPT // NBUF

    @pl.loop(0, G - 1)
    def _(k):
        j0 = k * NBUF
        for b in range(NBUF):
            gather(j0 + b, b).wait()
            scatter(j0 + b, b).start(add=True)
        for b in range(NBUF):
            scatter(j0 + b, b).wait()
            gather(j0 + NBUF + b, b).start()

    j0 = (G - 1) * NBUF
    for b in range(NBUF):
        gather(j0 + b, b).wait()
        scatter(j0 + b, b).start(add=True)
    for b in range(NBUF):
        scatter(j0 + b, b).wait()

    plsc.subcore_barrier()
    pltpu.sync_copy(
        acc.at[pl.ds(sid * RPT, RPT)],
        out_hbm.at[pl.ds(sid * RPT, RPT), pl.ds(cid * D, D)],
    )


def _make_prop(D):
    return functools.partial(
        pl.kernel,
        out_type=jax.ShapeDtypeStruct((NP, 2 * D), jnp.float32),
        mesh=_mesh,
        scratch_types=[
            pltpu.VMEM((CPT, CH), jnp.int32),
            pltpu.VMEM((CPT, CH), jnp.int32),
            pltpu.VMEM((NBUF, CH, D), jnp.float32),
            pltpu.MemorySpace.VMEM_SHARED((NP, D), jnp.float32),
        ]
        + [pltpu.SemaphoreType.DMA] * (2 * NBUF),
        compiler_params=_sc_params,
    )(functools.partial(_prop_body, D))


_prop64 = _make_prop(64)
_prop16 = _make_prop(16)


def _tc_first_body(x_ref, w_ref, degp_ref, xs_ref, dis_ref):
    degp = degp_ref[...]
    deg = degp[:, :1] + degp[:, 16:17] + 1.0
    dis = lax.rsqrt(deg)
    xw = jnp.dot(x_ref[...], w_ref[...], preferred_element_type=jnp.float32)
    xs_ref[...] = xw * dis
    dis_ref[...] = dis


def _tc_first(x, w1, degp):
    return pl.pallas_call(
        _tc_first_body,
        grid=(NP // ROW_BLK,),
        in_specs=[
            pl.BlockSpec((ROW_BLK, 128), lambda i: (i, 0)),
            pl.BlockSpec((128, 64), lambda i: (0, 0)),
            pl.BlockSpec((ROW_BLK, 32), lambda i: (i, 0)),
        ],
        out_specs=[
            pl.BlockSpec((ROW_BLK, 64), lambda i: (i, 0)),
            pl.BlockSpec((ROW_BLK, 1), lambda i: (i, 0)),
        ],
        out_shape=[
            jax.ShapeDtypeStruct((NP, 64), jnp.float32),
            jax.ShapeDtypeStruct((NP, 1), jnp.float32),
        ],
    )(x, w1, degp)


def _tc_mid_body(s_ref, xs_ref, dis_ref, b_ref, w_ref, out_ref):
    dis = dis_ref[...]
    s = s_ref[...]
    h = dis * (s[:, :64] + s[:, 64:] + xs_ref[...]) + b_ref[...]
    h = jnp.maximum(h, 0.0)
    xw = jnp.dot(h, w_ref[...], preferred_element_type=jnp.float32)
    out_ref[...] = xw * dis


def _tc_mid(s, xs, dis, b, w):
    dout = w.shape[1]
    return pl.pallas_call(
        _tc_mid_body,
        grid=(NP // ROW_BLK,),
        in_specs=[
            pl.BlockSpec((ROW_BLK, 128), lambda i: (i, 0)),
            pl.BlockSpec((ROW_BLK, 64), lambda i: (i, 0)),
            pl.BlockSpec((ROW_BLK, 1), lambda i: (i, 0)),
            pl.BlockSpec((1, 64), lambda i: (0, 0)),
            pl.BlockSpec((64, dout), lambda i: (0, 0)),
        ],
        out_specs=pl.BlockSpec((ROW_BLK, dout), lambda i: (i, 0)),
        out_shape=jax.ShapeDtypeStruct((NP, dout), jnp.float32),
    )(s, xs, dis, b, w)


def _tc_last_body(s_ref, xs_ref, dis_ref, b_ref, out_ref):
    s = s_ref[...]
    o = dis_ref[...] * (s[:, :16] + s[:, 16:] + xs_ref[...])
    out_ref[...] = o[:, :4] + b_ref[...]


def _tc_last(s, xs, dis, b3):
    return pl.pallas_call(
        _tc_last_body,
        grid=(NP // ROW_BLK,),
        in_specs=[
            pl.BlockSpec((ROW_BLK, 32), lambda i: (i, 0)),
            pl.BlockSpec((ROW_BLK, 16), lambda i: (i, 0)),
            pl.BlockSpec((ROW_BLK, 1), lambda i: (i, 0)),
            pl.BlockSpec((1, 4), lambda i: (0, 0)),
        ],
        out_specs=pl.BlockSpec((ROW_BLK, 4), lambda i: (i, 0)),
        out_shape=jax.ShapeDtypeStruct((NP, 4), jnp.float32),
    )(s, xs, dis, b3)


def kernel(x, edge_index, W1, b1, W2, b2, W3, b3):
    er = edge_index.astype(jnp.int32).reshape(2, NCHUNK, CH)

    ones125 = jnp.ones((CH, 16), jnp.float32)
    z16 = jnp.zeros((RPT, 16), jnp.float32)
    z64 = jnp.zeros((RPT, 64), jnp.float32)

    xp = jnp.pad(x, ((0, NP - N), (0, 0)))
    degp = _deg_kernel(er, ones125, z16)

    xs1, dis = _tc_first(xp, W1, degp)
    s1 = _prop64(xs1, er, z64)

    xs2 = _tc_mid(s1, xs1, dis, b1.reshape(1, 64), W2)
    s2 = _prop64(xs2, er, z64)

    w3p = jnp.pad(W3, ((0, 0), (0, 16 - W3.shape[1])))
    xs3 = _tc_mid(s2, xs2, dis, b2.reshape(1, 64), w3p)
    s3 = _prop16(xs3, er, z16)

    return _tc_last(s3, xs3, dis, b3.reshape(1, 4))[:N]
